# baseline (speedup 1.0000x reference)
import jax
import jax.numpy as jnp
from jax import lax
from jax.experimental import pallas as pl
from jax.experimental.pallas import tpu as pltpu
from jax.experimental.pallas import tpu_sc as plsc

D = 64
NC, NS = 2, 16
NW = NC * NS
C = 128
SCALE = 8.0


def _emb_body(xt_hbm, table_hbm, out_hbm, idx_v, gbufs, obufs, gsems, osems):
    nbuf = len(gbufs)
    S = xt_hbm.shape[0]
    wid = lax.axis_index("s") * NC + lax.axis_index("c")
    b0 = wid * C
    pltpu.sync_copy(xt_hbm.at[:, pl.ds(b0, C)], idx_v)

    lanes = lax.iota(jnp.int32, 16)
    rows = [lanes + g * 16 for g in range(C // 16)]

    for b in range(nbuf):
        pltpu.async_copy(table_hbm.at[idx_v.at[b]], gbufs[b], gsems[b])

    @pl.loop(0, S, step=nbuf)
    def _chunk(g):
        for b in range(nbuf):
            j = g + b
            pltpu.make_async_copy(
                table_hbm.at[idx_v.at[j]], gbufs[b], gsems[b]).wait()

            @pl.when(j >= nbuf)
            def _():
                pltpu.make_async_copy(
                    obufs[b],
                    out_hbm.at[j - nbuf, :, pl.ds(b0, C)],
                    osems[b]).wait()

            @pl.loop(0, D, step=2)
            def _d(d):
                for u in range(2):
                    dv = jnp.full((16,), d + u, jnp.int32)
                    for g2 in range(C // 16):
                        v = plsc.load_gather(gbufs[b], [rows[g2], dv])
                        obufs[b][d + u, pl.ds(g2 * 16, 16)] = v * SCALE

            @pl.when(j + nbuf < S)
            def _():
                pltpu.async_copy(
                    table_hbm.at[idx_v.at[j + nbuf]], gbufs[b], gsems[b])

            pltpu.async_copy(
                obufs[b], out_hbm.at[j, :, pl.ds(b0, C)], osems[b])

    for b in range(nbuf):
        pltpu.make_async_copy(
            obufs[b],
            out_hbm.at[S - nbuf + b, :, pl.ds(b0, C)],
            osems[b]).wait()


def kernel(x, table):
    B, S = x.shape
    assert B == NW * C
    xt = x.T.astype(jnp.int32)

    nbuf = 4
    mesh = plsc.VectorSubcoreMesh(core_axis_name="c", subcore_axis_name="s")
    k = pl.kernel(
        _emb_body,
        out_type=jax.ShapeDtypeStruct((S, D, B), jnp.float32),
        mesh=mesh,
        compiler_params=pltpu.CompilerParams(
            use_tc_tiling_on_sc=False, needs_layout_passes=False),
        scratch_types=[
            pltpu.VMEM((S, C), jnp.int32),
            [pltpu.VMEM((C, D), jnp.float32) for _ in range(nbuf)],
            [pltpu.VMEM((D, C), jnp.float32) for _ in range(nbuf)],
            [pltpu.SemaphoreType.DMA for _ in range(nbuf)],
            [pltpu.SemaphoreType.DMA for _ in range(nbuf)],
        ],
    )
    out = k(xt, table)
    return out.transpose(2, 0, 1)


# speedup vs baseline: 1.5833x; 1.5833x over previous
import jax
import jax.numpy as jnp
from jax import lax
from jax.experimental import pallas as pl
from jax.experimental.pallas import tpu as pltpu
from jax.experimental.pallas import tpu_sc as plsc

D = 64
NC, NS = 2, 16
NW = NC * NS
C = 128
SCALE = 8.0


def _emb_body(xt_hbm, table_hbm, out_hbm, idx_v, gbufs, obufs, gsems, osems):
    nbuf = len(gbufs)
    S = xt_hbm.shape[0]
    wid = lax.axis_index("s") * NC + lax.axis_index("c")
    b0 = wid * C
    pltpu.sync_copy(xt_hbm.at[:, pl.ds(b0, C)], idx_v)

    lanes = lax.iota(jnp.int32, 16)
    drows = [lanes + g * 16 for g in range(D // 16)]

    for b in range(nbuf):
        pltpu.async_copy(table_hbm.at[idx_v.at[b]], gbufs[b], gsems[b])

    @pl.loop(0, S, step=nbuf)
    def _chunk(g):
        for b in range(nbuf):
            j = g + b
            pltpu.make_async_copy(
                table_hbm.at[idx_v.at[j]], gbufs[b], gsems[b]).wait()

            @pl.when(j >= nbuf)
            def _():
                pltpu.make_async_copy(
                    obufs[b].at[:, pl.ds(0, C)],
                    out_hbm.at[j - nbuf, :, pl.ds(b0, C)],
                    osems[b]).wait()

            @pl.loop(0, C, step=2)
            def _r(r):
                for u in range(2):
                    rv = jnp.full((16,), r + u, jnp.int32)
                    for g2 in range(D // 16):
                        v = gbufs[b][r + u, pl.ds(g2 * 16, 16)]
                        plsc.store_scatter(
                            obufs[b], [drows[g2], rv], v * SCALE)

            @pl.when(j + nbuf < S)
            def _():
                pltpu.async_copy(
                    table_hbm.at[idx_v.at[j + nbuf]], gbufs[b], gsems[b])

            pltpu.async_copy(
                obufs[b].at[:, pl.ds(0, C)],
                out_hbm.at[j, :, pl.ds(b0, C)], osems[b])

    for b in range(nbuf):
        pltpu.make_async_copy(
            obufs[b].at[:, pl.ds(0, C)],
            out_hbm.at[S - nbuf + b, :, pl.ds(b0, C)],
            osems[b]).wait()


def kernel(x, table):
    B, S = x.shape
    assert B == NW * C
    xt = x.T.astype(jnp.int32)

    nbuf = 4
    mesh = plsc.VectorSubcoreMesh(core_axis_name="c", subcore_axis_name="s")
    k = pl.kernel(
        _emb_body,
        out_type=jax.ShapeDtypeStruct((S, D, B), jnp.float32),
        mesh=mesh,
        compiler_params=pltpu.CompilerParams(
            use_tc_tiling_on_sc=False, needs_layout_passes=False),
        scratch_types=[
            pltpu.VMEM((S, C), jnp.int32),
            [pltpu.VMEM((C, D), jnp.float32) for _ in range(nbuf)],
            [pltpu.VMEM((D, C + 1), jnp.float32) for _ in range(nbuf)],
            [pltpu.SemaphoreType.DMA for _ in range(nbuf)],
            [pltpu.SemaphoreType.DMA for _ in range(nbuf)],
        ],
    )
    out = k(xt, table)
    return out.transpose(2, 0, 1)
